# baseline (speedup 1.0000x reference)
import functools

import jax
import jax.numpy as jnp
import numpy as np
from jax import lax
from jax.experimental import pallas as pl
from jax.experimental.pallas import tpu as pltpu
from jax.experimental.pallas import tpu_sc as plsc

NUM_RBF = 16
NUM_POS = 16
TOP_K = 48
HIDDEN = 128
MAX_REL = 32
_CB_A = -0.58273431
_CB_B = 0.56802827
_CB_C = -0.54067466
RBF_D_MIN = 2.0
RBF_D_MAX = 22.0

_RBLK = 256
_RES = 64
_NW = 32

_NPAIR = 25
_F = 512
_FD = 384


def _pair_col(p, m_lo, t=0):
    return t * 128 + m_lo * 32 + p


def _prep_body(p_ref, mc_ref, tab_ref, tabc_ref):
    x = p_ref[0]
    c = lambda i: x[:, i:i + 1]
    bx = c(3) - c(0)
    by = c(4) - c(1)
    bz = c(5) - c(2)
    cx = c(6) - c(3)
    cy = c(7) - c(4)
    cz = c(8) - c(5)
    ax = by * cz - bz * cy
    ay = bz * cx - bx * cz
    az = bx * cy - by * cx
    cbx = _CB_A * ax + _CB_B * bx + _CB_C * cx + c(3)
    cby = _CB_A * ay + _CB_B * by + _CB_C * cy + c(4)
    cbz = _CB_A * az + _CB_B * bz + _CB_C * cz + c(5)
    zeros = jnp.zeros((x.shape[0], 15), jnp.float32)
    base = jnp.concatenate(
        [x[:, 0:12], cbx, cby, cbz, c(12), c(13)], axis=1)
    tab_ref[0] = jnp.concatenate([base, zeros], axis=1)
    tabc_ref[0] = jnp.dot(base, mc_ref[...],
                          preferred_element_type=jnp.float32,
                          precision=lax.Precision.HIGHEST)


def _topk_body(xc_ref, xr_ref, m_ref, eidx_ref, flat_ref, d_ref, *, L):
    b = pl.program_id(0)
    xc = xc_ref[0]
    xr = xr_ref[0]
    m = m_ref[0]
    dx = xc[:, 3:4] - xr[3:4, :]
    dy = xc[:, 4:5] - xr[4:5, :]
    dz = xc[:, 5:6] - xr[5:6, :]
    d = dx * dx + dy * dy + dz * dz
    d = d + (1.0 - m) * 1000000.0
    d_ref[...] = d
    kiota = lax.broadcasted_iota(jnp.int32, (_RBLK, TOP_K), 1)

    def body(k, carry):
        mnp, argp, eidx = carry
        dd = d_ref[...]
        iota = lax.broadcasted_iota(jnp.int32, (_RBLK, L), 1)
        igt = iota > argp
        rem = (dd > mnp) | ((dd == mnp) & igt)
        mn = jnp.min(jnp.where(rem, dd, jnp.float32(1e30)), axis=1,
                     keepdims=True)
        cand = (dd == mn) & ((mn > mnp) | igt)
        arg = jnp.min(jnp.where(cand, iota, jnp.int32(2**30)), axis=1,
                      keepdims=True)
        eidx = jnp.where(kiota == k, arg, eidx)
        return mn, arg, eidx

    init = (jnp.full((_RBLK, 1), -1.0, jnp.float32),
            jnp.full((_RBLK, 1), -1, jnp.int32),
            jnp.zeros((_RBLK, TOP_K), jnp.int32))
    _, _, eidx = lax.fori_loop(0, TOP_K, body, init)
    eidx_ref[0] = eidx
    flat_ref[0] = eidx + b * L


def _gather_body(idx_hbm, tab_hbm, out_hbm, idx_v, rows_v, sem, *, cpw):
    wid = lax.axis_index("s") * 2 + lax.axis_index("c")
    base = wid * cpw
    pltpu.sync_copy(idx_hbm.at[pl.ds(base, cpw)], idx_v)

    def body(j, carry):
        pltpu.async_copy(tab_hbm.at[idx_v.at[j]], rows_v.at[j], sem).wait()
        return carry

    lax.fori_loop(0, cpw, body, 0)
    pltpu.sync_copy(rows_v, out_hbm.at[pl.ds(base, cpw)])


def _gather_rows(idx2d, tab2d):
    nrows = idx2d.shape[0]
    cpw = nrows // _NW
    mesh = plsc.VectorSubcoreMesh(
        core_axis_name="c", subcore_axis_name="s", num_cores=2, num_subcores=16)
    fn = pl.kernel(
        functools.partial(_gather_body, cpw=cpw),
        out_type=jax.ShapeDtypeStruct((nrows, 128, 32), jnp.float32),
        mesh=mesh,
        scratch_types=[
            pltpu.VMEM((cpw, 128), jnp.int32),
            pltpu.VMEM((cpw, 128, 32), jnp.float32),
            pltpu.SemaphoreType.DMA,
        ],
        compiler_params=pltpu.CompilerParams(use_tc_tiling_on_sc=False),
    )
    return fn(idx2d, tab2d)


def _feat_body(tabc_ref, gat_ref, rmat_ref, negmn_ref, mu_ref,
               mpos_ref, w2t_ref, bias_ref, gam_ref, bet_ref, out_ref):
    f32 = jnp.float32
    EB = _RES * TOP_K
    hc = tabc_ref[0]
    g = gat_ref[0].reshape(EB, 32)
    bf16 = jnp.bfloat16
    g_hi = g.astype(bf16)
    g_lo = (g - g_hi.astype(f32)).astype(bf16)
    hc_hi = hc.astype(bf16)
    hc_lo = (hc - hc_hi.astype(f32)).astype(bf16)
    rmat_b = rmat_ref[...]
    negmn_b = negmn_ref[...]
    lhs_hi = jnp.concatenate([rmat_b, g_hi], axis=1)
    lhs_lo = jnp.concatenate([rmat_b, g_lo], axis=1)
    rhs_hi = jnp.concatenate([hc_hi, negmn_b], axis=0)
    rhs_lo = jnp.concatenate([hc_lo, negmn_b], axis=0)
    diffp = (jnp.dot(lhs_hi, rhs_hi, preferred_element_type=f32)
             + jnp.dot(lhs_lo, rhs_lo, preferred_element_type=f32))
    dxp = diffp[:, 0:128]
    dyp = diffp[:, 128:256]
    dzp = diffp[:, 256:384]
    ssq = dxp * dxp + dyp * dyp + dzp * dzp
    dist = jnp.sqrt(ssq + 1e-06)
    d512 = jnp.concatenate([dist, dist, dist, dist], axis=1)
    z = d512 - mu_ref[...]
    rbf = jnp.exp(-(z * z) * 0.64)
    off = diffp[:, 26:27]
    ec = (diffp[:, 25:26] == 0.0).astype(f32)
    dpos = jnp.clip(off + 32.0, 0.0, 64.0) * ec + (1.0 - ec) * 65.0
    lane = lax.broadcasted_iota(jnp.int32, (EB, 128), 1).astype(f32)
    oneh = (lane == dpos).astype(f32)
    acc = (jnp.dot(oneh, mpos_ref[...], preferred_element_type=f32)
           + jnp.dot(rbf, w2t_ref[...], preferred_element_type=f32)
           + bias_ref[...])
    mu_ln = jnp.mean(acc, axis=1, keepdims=True)
    xcen = acc - mu_ln
    var = jnp.mean(xcen * xcen, axis=1, keepdims=True)
    y = xcen * lax.rsqrt(var + 1e-05) * gam_ref[...] + bet_ref[...]
    out_ref[0] = y.reshape(_RES, TOP_K, HIDDEN)


def kernel(X, residue_mask, R_idx, chain_labels, W_pos, b_pos, W_edge,
           gamma, beta):
    f32 = jnp.float32
    B, L = X.shape[0], X.shape[1]
    X = X.astype(f32)
    X2 = X.reshape(B, L, 12)
    Xt = jnp.transpose(X2, (0, 2, 1))
    maskr = residue_mask.astype(f32).reshape(B, 1, L)
    chain_f = chain_labels.astype(f32)[..., None]
    r_f = R_idx.astype(f32)[..., None]
    P = jnp.concatenate([X2, chain_f, r_f], axis=-1)

    mc = np.zeros((17, _FD), np.float32)
    negmn = np.zeros((32, _FD), np.float32)
    for t in range(3):
        for p in range(_NPAIR):
            for m_lo in range(4):
                mc[(p // 5) * 3 + t, _pair_col(p, m_lo, t)] = 1.0
                negmn[(p % 5) * 3 + t, _pair_col(p, m_lo, t)] = -1.0
    mc[15, 25] = 1.0
    mc[16, 26] = 1.0
    negmn[15, 25] = -1.0
    negmn[16, 26] = -1.0

    table, tabc = pl.pallas_call(
        _prep_body,
        grid=(B,),
        in_specs=[
            pl.BlockSpec((1, L, 14), lambda b: (b, 0, 0)),
            pl.BlockSpec((17, _FD), lambda b: (0, 0)),
        ],
        out_specs=[
            pl.BlockSpec((1, L, 32), lambda b: (b, 0, 0)),
            pl.BlockSpec((1, L, _FD), lambda b: (b, 0, 0)),
        ],
        out_shape=[
            jax.ShapeDtypeStruct((B, L, 32), f32),
            jax.ShapeDtypeStruct((B, L, _FD), f32),
        ],
    )(P, jnp.asarray(mc))

    nblk = L // _RBLK
    eidx, flat = pl.pallas_call(
        functools.partial(_topk_body, L=L),
        grid=(B, nblk),
        in_specs=[
            pl.BlockSpec((1, _RBLK, 12), lambda b, i: (b, i, 0)),
            pl.BlockSpec((1, 12, L), lambda b, i: (b, 0, 0)),
            pl.BlockSpec((1, 1, L), lambda b, i: (b, 0, 0)),
        ],
        out_specs=[
            pl.BlockSpec((1, _RBLK, TOP_K), lambda b, i: (b, i, 0)),
            pl.BlockSpec((1, _RBLK, TOP_K), lambda b, i: (b, i, 0)),
        ],
        out_shape=[
            jax.ShapeDtypeStruct((B, L, TOP_K), jnp.int32),
            jax.ShapeDtypeStruct((B, L, TOP_K), jnp.int32),
        ],
        scratch_shapes=[pltpu.VMEM((_RBLK, L), jnp.float32)],
    )(X2, Xt, maskr)

    idx2d = flat.reshape(B * L * TOP_K // 128, 128)
    tab2d = table.reshape(B * L, 32)
    gathered = _gather_rows(idx2d, tab2d).reshape(B, L, TOP_K, 32)

    EB = _RES * TOP_K
    e_ids = np.arange(EB) // TOP_K
    rmat = (e_ids[:, None] == np.arange(_RES)[None, :]).astype(np.float32)
    mu = np.linspace(RBF_D_MIN, RBF_D_MAX, NUM_RBF).astype(np.float32)
    mu512 = np.zeros((1, _F), np.float32)
    src_idx = np.zeros(_F, np.int32)
    valid = np.zeros(_F, np.bool_)
    for q in range(_F):
        p = q % 32
        m = (q // 128) * 4 + (q % 128) // 32
        if p < _NPAIR:
            mu512[0, q] = mu[m]
            src_idx[q] = p * NUM_RBF + m
            valid[q] = True
    w2t = jnp.where(jnp.asarray(valid)[:, None],
                    W_edge.astype(f32)[:, NUM_POS:].T[jnp.asarray(src_idx)],
                    0.0)
    W_pos = W_pos.astype(f32)
    W_edge = W_edge.astype(f32)
    mpos = jnp.zeros((128, HIDDEN), f32).at[0:66].set(
        W_pos.T @ W_edge[:, :NUM_POS].T)
    bias1 = (W_edge[:, :NUM_POS] @ b_pos.astype(f32))[None, :]
    gam = gamma.astype(f32)[None, :]
    bet = beta.astype(f32)[None, :]

    nres = L // _RES
    wspec = lambda shape: pl.BlockSpec(shape, lambda b, i: tuple(0 for _ in shape))
    E = pl.pallas_call(
        _feat_body,
        grid=(B, nres),
        in_specs=[
            pl.BlockSpec((1, _RES, _FD), lambda b, i: (b, i, 0)),
            pl.BlockSpec((1, _RES, TOP_K, 32), lambda b, i: (b, i, 0, 0)),
            wspec((EB, _RES)),
            wspec((32, _FD)),
            wspec((1, _F)),
            wspec((128, HIDDEN)),
            wspec((_F, HIDDEN)),
            wspec((1, HIDDEN)),
            wspec((1, HIDDEN)),
            wspec((1, HIDDEN)),
        ],
        out_specs=pl.BlockSpec((1, _RES, TOP_K, HIDDEN),
                               lambda b, i: (b, i, 0, 0)),
        out_shape=jax.ShapeDtypeStruct((B, L, TOP_K, HIDDEN), f32),
    )(tabc, gathered, jnp.asarray(rmat, jnp.bfloat16), jnp.asarray(negmn, jnp.bfloat16), jnp.asarray(mu512),
      mpos, w2t, bias1, gam, bet)

    return eidx, E


# speedup vs baseline: 1.2921x; 1.2921x over previous
import functools

import jax
import jax.numpy as jnp
import numpy as np
from jax import lax
from jax.experimental import pallas as pl
from jax.experimental.pallas import tpu as pltpu
from jax.experimental.pallas import tpu_sc as plsc

NUM_RBF = 16
NUM_POS = 16
TOP_K = 48
HIDDEN = 128
MAX_REL = 32
_CB_A = -0.58273431
_CB_B = 0.56802827
_CB_C = -0.54067466
RBF_D_MIN = 2.0
RBF_D_MAX = 22.0

_RBLK = 256
_RES = 64
_NW = 32

_NPAIR = 25
_F = 512
_FD = 384


def _pair_col(p, m_lo, t=0):
    return t * 128 + m_lo * 32 + p


def _prep_body(p_ref, mc_ref, tab_ref, tabc_ref):
    x = p_ref[0]
    c = lambda i: x[:, i:i + 1]
    bx = c(3) - c(0)
    by = c(4) - c(1)
    bz = c(5) - c(2)
    cx = c(6) - c(3)
    cy = c(7) - c(4)
    cz = c(8) - c(5)
    ax = by * cz - bz * cy
    ay = bz * cx - bx * cz
    az = bx * cy - by * cx
    cbx = _CB_A * ax + _CB_B * bx + _CB_C * cx + c(3)
    cby = _CB_A * ay + _CB_B * by + _CB_C * cy + c(4)
    cbz = _CB_A * az + _CB_B * bz + _CB_C * cz + c(5)
    zeros = jnp.zeros((x.shape[0], 15), jnp.float32)
    base = jnp.concatenate(
        [x[:, 0:12], cbx, cby, cbz, c(12), c(13)], axis=1)
    tab_ref[0] = jnp.concatenate([base, zeros], axis=1)
    tabc_ref[0] = jnp.dot(base, mc_ref[...],
                          preferred_element_type=jnp.float32,
                          precision=lax.Precision.HIGHEST)


def _topk_body(xc_ref, xr_ref, m_ref, eidx_ref, flat_ref, d_ref, *, L):
    b = pl.program_id(0)
    xc = xc_ref[0]
    xr = xr_ref[0]
    m = m_ref[0]
    dx = xc[:, 3:4] - xr[3:4, :]
    dy = xc[:, 4:5] - xr[4:5, :]
    dz = xc[:, 5:6] - xr[5:6, :]
    d = dx * dx + dy * dy + dz * dz
    d = d + (1.0 - m) * 1000000.0
    d_ref[...] = d
    kiota = lax.broadcasted_iota(jnp.int32, (_RBLK, TOP_K), 1)

    def body(k, eidx):
        dcur = d_ref[...]
        iota = lax.broadcasted_iota(jnp.int32, (_RBLK, L), 1)
        mn = jnp.min(dcur, axis=1, keepdims=True)
        sel = jnp.where(dcur == mn, iota, jnp.int32(2**30))
        arg = jnp.min(sel, axis=1, keepdims=True)
        eidx = jnp.where(kiota == k, arg, eidx)
        d_ref[...] = jnp.where(iota == arg, jnp.float32(1e30), dcur)
        return eidx

    eidx = lax.fori_loop(0, TOP_K, body,
                         jnp.zeros((_RBLK, TOP_K), jnp.int32))
    eidx_ref[0] = eidx
    flat_ref[0] = eidx + b * L


def _gather_body(idx_hbm, tab_hbm, out_hbm, idx_v, rows_v, sem, *, cpw):
    wid = lax.axis_index("s") * 2 + lax.axis_index("c")
    base = wid * cpw
    pltpu.sync_copy(idx_hbm.at[pl.ds(base, cpw)], idx_v)

    def body(j, carry):
        pltpu.async_copy(tab_hbm.at[idx_v.at[j]], rows_v.at[j], sem).wait()
        return carry

    lax.fori_loop(0, cpw, body, 0)
    pltpu.sync_copy(rows_v, out_hbm.at[pl.ds(base, cpw)])


def _gather_rows(idx2d, tab2d):
    nrows = idx2d.shape[0]
    cpw = nrows // _NW
    mesh = plsc.VectorSubcoreMesh(
        core_axis_name="c", subcore_axis_name="s", num_cores=2, num_subcores=16)
    fn = pl.kernel(
        functools.partial(_gather_body, cpw=cpw),
        out_type=jax.ShapeDtypeStruct((nrows, 128, 32), jnp.float32),
        mesh=mesh,
        scratch_types=[
            pltpu.VMEM((cpw, 128), jnp.int32),
            pltpu.VMEM((cpw, 128, 32), jnp.float32),
            pltpu.SemaphoreType.DMA,
        ],
        compiler_params=pltpu.CompilerParams(use_tc_tiling_on_sc=False),
    )
    return fn(idx2d, tab2d)


def _feat_body(tabc_ref, gat_ref, rmat_ref, negmn_ref, mu_ref,
               mpos_ref, w2t_ref, bias_ref, gam_ref, bet_ref, out_ref):
    f32 = jnp.float32
    EB = _RES * TOP_K
    hc = tabc_ref[0]
    g = gat_ref[0].reshape(EB, 32)
    bf16 = jnp.bfloat16
    g_hi = g.astype(bf16)
    g_lo = (g - g_hi.astype(f32)).astype(bf16)
    hc_hi = hc.astype(bf16)
    hc_lo = (hc - hc_hi.astype(f32)).astype(bf16)
    rmat_b = rmat_ref[...]
    negmn_b = negmn_ref[...]
    lhs_hi = jnp.concatenate([rmat_b, g_hi], axis=1)
    lhs_lo = jnp.concatenate([rmat_b, g_lo], axis=1)
    rhs_hi = jnp.concatenate([hc_hi, negmn_b], axis=0)
    rhs_lo = jnp.concatenate([hc_lo, negmn_b], axis=0)
    diffp = (jnp.dot(lhs_hi, rhs_hi, preferred_element_type=f32)
             + jnp.dot(lhs_lo, rhs_lo, preferred_element_type=f32))
    dxp = diffp[:, 0:128]
    dyp = diffp[:, 128:256]
    dzp = diffp[:, 256:384]
    ssq = dxp * dxp + dyp * dyp + dzp * dzp
    dist = jnp.sqrt(ssq + 1e-06)
    d512 = jnp.concatenate([dist, dist, dist, dist], axis=1)
    z = d512 - mu_ref[...]
    rbf = jnp.exp(-(z * z) * 0.64)
    off = diffp[:, 26:27]
    ec = (diffp[:, 25:26] == 0.0).astype(f32)
    dpos = jnp.clip(off + 32.0, 0.0, 64.0) * ec + (1.0 - ec) * 65.0
    lane = lax.broadcasted_iota(jnp.int32, (EB, 128), 1).astype(f32)
    oneh = (lane == dpos).astype(f32)
    acc = (jnp.dot(oneh, mpos_ref[...], preferred_element_type=f32)
           + jnp.dot(rbf, w2t_ref[...], preferred_element_type=f32)
           + bias_ref[...])
    mu_ln = jnp.mean(acc, axis=1, keepdims=True)
    xcen = acc - mu_ln
    var = jnp.mean(xcen * xcen, axis=1, keepdims=True)
    y = xcen * lax.rsqrt(var + 1e-05) * gam_ref[...] + bet_ref[...]
    out_ref[0] = y.reshape(_RES, TOP_K, HIDDEN)


def kernel(X, residue_mask, R_idx, chain_labels, W_pos, b_pos, W_edge,
           gamma, beta):
    f32 = jnp.float32
    B, L = X.shape[0], X.shape[1]
    X = X.astype(f32)
    X2 = X.reshape(B, L, 12)
    Xt = jnp.transpose(X2, (0, 2, 1))
    maskr = residue_mask.astype(f32).reshape(B, 1, L)
    chain_f = chain_labels.astype(f32)[..., None]
    r_f = R_idx.astype(f32)[..., None]
    P = jnp.concatenate([X2, chain_f, r_f], axis=-1)

    mc = np.zeros((17, _FD), np.float32)
    negmn = np.zeros((32, _FD), np.float32)
    for t in range(3):
        for p in range(_NPAIR):
            for m_lo in range(4):
                mc[(p // 5) * 3 + t, _pair_col(p, m_lo, t)] = 1.0
                negmn[(p % 5) * 3 + t, _pair_col(p, m_lo, t)] = -1.0
    mc[15, 25] = 1.0
    mc[16, 26] = 1.0
    negmn[15, 25] = -1.0
    negmn[16, 26] = -1.0

    table, tabc = pl.pallas_call(
        _prep_body,
        grid=(B,),
        in_specs=[
            pl.BlockSpec((1, L, 14), lambda b: (b, 0, 0)),
            pl.BlockSpec((17, _FD), lambda b: (0, 0)),
        ],
        out_specs=[
            pl.BlockSpec((1, L, 32), lambda b: (b, 0, 0)),
            pl.BlockSpec((1, L, _FD), lambda b: (b, 0, 0)),
        ],
        out_shape=[
            jax.ShapeDtypeStruct((B, L, 32), f32),
            jax.ShapeDtypeStruct((B, L, _FD), f32),
        ],
    )(P, jnp.asarray(mc))

    nblk = L // _RBLK
    eidx, flat = pl.pallas_call(
        functools.partial(_topk_body, L=L),
        grid=(B, nblk),
        in_specs=[
            pl.BlockSpec((1, _RBLK, 12), lambda b, i: (b, i, 0)),
            pl.BlockSpec((1, 12, L), lambda b, i: (b, 0, 0)),
            pl.BlockSpec((1, 1, L), lambda b, i: (b, 0, 0)),
        ],
        out_specs=[
            pl.BlockSpec((1, _RBLK, TOP_K), lambda b, i: (b, i, 0)),
            pl.BlockSpec((1, _RBLK, TOP_K), lambda b, i: (b, i, 0)),
        ],
        out_shape=[
            jax.ShapeDtypeStruct((B, L, TOP_K), jnp.int32),
            jax.ShapeDtypeStruct((B, L, TOP_K), jnp.int32),
        ],
        scratch_shapes=[pltpu.VMEM((_RBLK, L), jnp.float32)],
    )(X2, Xt, maskr)

    idx2d = flat.reshape(B * L * TOP_K // 128, 128)
    tab2d = table.reshape(B * L, 32)
    gathered = _gather_rows(idx2d, tab2d).reshape(B, L, TOP_K, 32)

    EB = _RES * TOP_K
    e_ids = np.arange(EB) // TOP_K
    rmat = (e_ids[:, None] == np.arange(_RES)[None, :]).astype(np.float32)
    mu = np.linspace(RBF_D_MIN, RBF_D_MAX, NUM_RBF).astype(np.float32)
    mu512 = np.zeros((1, _F), np.float32)
    src_idx = np.zeros(_F, np.int32)
    valid = np.zeros(_F, np.bool_)
    for q in range(_F):
        p = q % 32
        m = (q // 128) * 4 + (q % 128) // 32
        if p < _NPAIR:
            mu512[0, q] = mu[m]
            src_idx[q] = p * NUM_RBF + m
            valid[q] = True
    w2t = jnp.where(jnp.asarray(valid)[:, None],
                    W_edge.astype(f32)[:, NUM_POS:].T[jnp.asarray(src_idx)],
                    0.0)
    W_pos = W_pos.astype(f32)
    W_edge = W_edge.astype(f32)
    mpos = jnp.zeros((128, HIDDEN), f32).at[0:66].set(
        W_pos.T @ W_edge[:, :NUM_POS].T)
    bias1 = (W_edge[:, :NUM_POS] @ b_pos.astype(f32))[None, :]
    gam = gamma.astype(f32)[None, :]
    bet = beta.astype(f32)[None, :]

    nres = L // _RES
    wspec = lambda shape: pl.BlockSpec(shape, lambda b, i: tuple(0 for _ in shape))
    E = pl.pallas_call(
        _feat_body,
        grid=(B, nres),
        in_specs=[
            pl.BlockSpec((1, _RES, _FD), lambda b, i: (b, i, 0)),
            pl.BlockSpec((1, _RES, TOP_K, 32), lambda b, i: (b, i, 0, 0)),
            wspec((EB, _RES)),
            wspec((32, _FD)),
            wspec((1, _F)),
            wspec((128, HIDDEN)),
            wspec((_F, HIDDEN)),
            wspec((1, HIDDEN)),
            wspec((1, HIDDEN)),
            wspec((1, HIDDEN)),
        ],
        out_specs=pl.BlockSpec((1, _RES, TOP_K, HIDDEN),
                               lambda b, i: (b, i, 0, 0)),
        out_shape=jax.ShapeDtypeStruct((B, L, TOP_K, HIDDEN), f32),
    )(tabc, gathered, jnp.asarray(rmat, jnp.bfloat16), jnp.asarray(negmn, jnp.bfloat16), jnp.asarray(mu512),
      mpos, w2t, bias1, gam, bet)

    return eidx, E


# speedup vs baseline: 1.3144x; 1.0173x over previous
import functools

import jax
import jax.numpy as jnp
import numpy as np
from jax import lax
from jax.experimental import pallas as pl
from jax.experimental.pallas import tpu as pltpu
from jax.experimental.pallas import tpu_sc as plsc

NUM_RBF = 16
NUM_POS = 16
TOP_K = 48
HIDDEN = 128
MAX_REL = 32
_CB_A = -0.58273431
_CB_B = 0.56802827
_CB_C = -0.54067466
RBF_D_MIN = 2.0
RBF_D_MAX = 22.0

_RBLK = 256
_RES = 64
_NW = 32

_NPAIR = 25
_F = 512
_FD = 384


def _pair_col(p, m_lo, t=0):
    return t * 128 + m_lo * 32 + p


def _prep_body(p_ref, mc_ref, tab_ref, tabc_ref):
    x = p_ref[0]
    c = lambda i: x[:, i:i + 1]
    bx = c(3) - c(0)
    by = c(4) - c(1)
    bz = c(5) - c(2)
    cx = c(6) - c(3)
    cy = c(7) - c(4)
    cz = c(8) - c(5)
    ax = by * cz - bz * cy
    ay = bz * cx - bx * cz
    az = bx * cy - by * cx
    cbx = _CB_A * ax + _CB_B * bx + _CB_C * cx + c(3)
    cby = _CB_A * ay + _CB_B * by + _CB_C * cy + c(4)
    cbz = _CB_A * az + _CB_B * bz + _CB_C * cz + c(5)
    zeros = jnp.zeros((x.shape[0], 15), jnp.float32)
    base = jnp.concatenate(
        [x[:, 0:12], cbx, cby, cbz, c(12), c(13)], axis=1)
    tab_ref[0] = jnp.concatenate([base, zeros], axis=1)
    tabc_ref[0] = jnp.dot(base, mc_ref[...],
                          preferred_element_type=jnp.float32,
                          precision=lax.Precision.HIGHEST)


def _topk_body(xc_ref, xr_ref, m_ref, eidx_ref, flat_ref, d_ref, *, L):
    b = pl.program_id(0)
    xc = xc_ref[0]
    xr = xr_ref[0]
    m = m_ref[0]
    dx = xc[:, 3:4] - xr[3:4, :]
    dy = xc[:, 4:5] - xr[4:5, :]
    dz = xc[:, 5:6] - xr[5:6, :]
    d = dx * dx + dy * dy + dz * dz
    d = d + (1.0 - m) * 1000000.0
    d_ref[...] = d
    kiota = lax.broadcasted_iota(jnp.int32, (_RBLK, TOP_K), 1)

    def body(k, eidx):
        dcur = d_ref[...]
        iota = lax.broadcasted_iota(jnp.int32, (_RBLK, L), 1)
        mn = jnp.min(dcur, axis=1, keepdims=True)
        sel = jnp.where(dcur == mn, iota, jnp.int32(2**30))
        arg = jnp.min(sel, axis=1, keepdims=True)
        eidx = jnp.where(kiota == k, arg, eidx)
        d_ref[...] = jnp.where(iota == arg, jnp.float32(1e30), dcur)
        return eidx

    eidx = lax.fori_loop(0, TOP_K, body,
                         jnp.zeros((_RBLK, TOP_K), jnp.int32))
    eidx_ref[0] = eidx
    flat_ref[0] = eidx + b * L


def _gather_body(idx_hbm, tab_hbm, out_hbm, idx_v, rows_v, sem, *, cpw):
    wid = lax.axis_index("s") * 2 + lax.axis_index("c")
    base = wid * cpw
    pltpu.sync_copy(idx_hbm.at[pl.ds(base, cpw)], idx_v)

    def fire(j, carry):
        pltpu.async_copy(tab_hbm.at[idx_v.at[j]], rows_v.at[j], sem)
        return carry

    def drain(j, carry):
        pltpu.make_async_copy(tab_hbm.at[idx_v.at[j]], rows_v.at[j], sem).wait()
        return carry

    lax.fori_loop(0, cpw, fire, 0)
    lax.fori_loop(0, cpw, drain, 0)
    pltpu.sync_copy(rows_v, out_hbm.at[pl.ds(base, cpw)])


def _gather_rows(idx2d, tab2d):
    nrows = idx2d.shape[0]
    cpw = nrows // _NW
    mesh = plsc.VectorSubcoreMesh(
        core_axis_name="c", subcore_axis_name="s", num_cores=2, num_subcores=16)
    fn = pl.kernel(
        functools.partial(_gather_body, cpw=cpw),
        out_type=jax.ShapeDtypeStruct((nrows, 128, 32), jnp.float32),
        mesh=mesh,
        scratch_types=[
            pltpu.VMEM((cpw, 128), jnp.int32),
            pltpu.VMEM((cpw, 128, 32), jnp.float32),
            pltpu.SemaphoreType.DMA,
        ],
        compiler_params=pltpu.CompilerParams(use_tc_tiling_on_sc=False),
    )
    return fn(idx2d, tab2d)


def _feat_body(tabc_ref, gat_ref, rmat_ref, negmn_ref, mu_ref,
               mpos_ref, w2t_ref, bias_ref, gam_ref, bet_ref, out_ref):
    f32 = jnp.float32
    EB = _RES * TOP_K
    hc = tabc_ref[0]
    g = gat_ref[0].reshape(EB, 32)
    bf16 = jnp.bfloat16
    g_hi = g.astype(bf16)
    g_lo = (g - g_hi.astype(f32)).astype(bf16)
    hc_hi = hc.astype(bf16)
    hc_lo = (hc - hc_hi.astype(f32)).astype(bf16)
    rmat_b = rmat_ref[...]
    negmn_b = negmn_ref[...]
    lhs_hi = jnp.concatenate([rmat_b, g_hi], axis=1)
    lhs_lo = jnp.concatenate([rmat_b, g_lo], axis=1)
    rhs_hi = jnp.concatenate([hc_hi, negmn_b], axis=0)
    rhs_lo = jnp.concatenate([hc_lo, negmn_b], axis=0)
    diffp = (jnp.dot(lhs_hi, rhs_hi, preferred_element_type=f32)
             + jnp.dot(lhs_lo, rhs_lo, preferred_element_type=f32))
    dxp = diffp[:, 0:128]
    dyp = diffp[:, 128:256]
    dzp = diffp[:, 256:384]
    ssq = dxp * dxp + dyp * dyp + dzp * dzp
    dist = jnp.sqrt(ssq + 1e-06)
    d512 = jnp.concatenate([dist, dist, dist, dist], axis=1)
    z = d512 - mu_ref[...]
    rbf = jnp.exp(-(z * z) * 0.64)
    off = diffp[:, 26:27]
    ec = (diffp[:, 25:26] == 0.0).astype(f32)
    dpos = jnp.clip(off + 32.0, 0.0, 64.0) * ec + (1.0 - ec) * 65.0
    lane = lax.broadcasted_iota(jnp.int32, (EB, 128), 1).astype(f32)
    oneh = (lane == dpos).astype(f32)
    acc = (jnp.dot(oneh, mpos_ref[...], preferred_element_type=f32)
           + jnp.dot(rbf, w2t_ref[...], preferred_element_type=f32)
           + bias_ref[...])
    mu_ln = jnp.mean(acc, axis=1, keepdims=True)
    xcen = acc - mu_ln
    var = jnp.mean(xcen * xcen, axis=1, keepdims=True)
    y = xcen * lax.rsqrt(var + 1e-05) * gam_ref[...] + bet_ref[...]
    out_ref[0] = y.reshape(_RES, TOP_K, HIDDEN)


def kernel(X, residue_mask, R_idx, chain_labels, W_pos, b_pos, W_edge,
           gamma, beta):
    f32 = jnp.float32
    B, L = X.shape[0], X.shape[1]
    X = X.astype(f32)
    X2 = X.reshape(B, L, 12)
    Xt = jnp.transpose(X2, (0, 2, 1))
    maskr = residue_mask.astype(f32).reshape(B, 1, L)
    chain_f = chain_labels.astype(f32)[..., None]
    r_f = R_idx.astype(f32)[..., None]
    P = jnp.concatenate([X2, chain_f, r_f], axis=-1)

    mc = np.zeros((17, _FD), np.float32)
    negmn = np.zeros((32, _FD), np.float32)
    for t in range(3):
        for p in range(_NPAIR):
            for m_lo in range(4):
                mc[(p // 5) * 3 + t, _pair_col(p, m_lo, t)] = 1.0
                negmn[(p % 5) * 3 + t, _pair_col(p, m_lo, t)] = -1.0
    mc[15, 25] = 1.0
    mc[16, 26] = 1.0
    negmn[15, 25] = -1.0
    negmn[16, 26] = -1.0

    table, tabc = pl.pallas_call(
        _prep_body,
        grid=(B,),
        in_specs=[
            pl.BlockSpec((1, L, 14), lambda b: (b, 0, 0)),
            pl.BlockSpec((17, _FD), lambda b: (0, 0)),
        ],
        out_specs=[
            pl.BlockSpec((1, L, 32), lambda b: (b, 0, 0)),
            pl.BlockSpec((1, L, _FD), lambda b: (b, 0, 0)),
        ],
        out_shape=[
            jax.ShapeDtypeStruct((B, L, 32), f32),
            jax.ShapeDtypeStruct((B, L, _FD), f32),
        ],
    )(P, jnp.asarray(mc))

    nblk = L // _RBLK
    eidx, flat = pl.pallas_call(
        functools.partial(_topk_body, L=L),
        grid=(B, nblk),
        in_specs=[
            pl.BlockSpec((1, _RBLK, 12), lambda b, i: (b, i, 0)),
            pl.BlockSpec((1, 12, L), lambda b, i: (b, 0, 0)),
            pl.BlockSpec((1, 1, L), lambda b, i: (b, 0, 0)),
        ],
        out_specs=[
            pl.BlockSpec((1, _RBLK, TOP_K), lambda b, i: (b, i, 0)),
            pl.BlockSpec((1, _RBLK, TOP_K), lambda b, i: (b, i, 0)),
        ],
        out_shape=[
            jax.ShapeDtypeStruct((B, L, TOP_K), jnp.int32),
            jax.ShapeDtypeStruct((B, L, TOP_K), jnp.int32),
        ],
        scratch_shapes=[pltpu.VMEM((_RBLK, L), jnp.float32)],
    )(X2, Xt, maskr)

    idx2d = flat.reshape(B * L * TOP_K // 128, 128)
    tab2d = table.reshape(B * L, 32)
    gathered = _gather_rows(idx2d, tab2d).reshape(B, L, TOP_K, 32)

    EB = _RES * TOP_K
    e_ids = np.arange(EB) // TOP_K
    rmat = (e_ids[:, None] == np.arange(_RES)[None, :]).astype(np.float32)
    mu = np.linspace(RBF_D_MIN, RBF_D_MAX, NUM_RBF).astype(np.float32)
    mu512 = np.zeros((1, _F), np.float32)
    src_idx = np.zeros(_F, np.int32)
    valid = np.zeros(_F, np.bool_)
    for q in range(_F):
        p = q % 32
        m = (q // 128) * 4 + (q % 128) // 32
        if p < _NPAIR:
            mu512[0, q] = mu[m]
            src_idx[q] = p * NUM_RBF + m
            valid[q] = True
    w2t = jnp.where(jnp.asarray(valid)[:, None],
                    W_edge.astype(f32)[:, NUM_POS:].T[jnp.asarray(src_idx)],
                    0.0)
    W_pos = W_pos.astype(f32)
    W_edge = W_edge.astype(f32)
    mpos = jnp.zeros((128, HIDDEN), f32).at[0:66].set(
        W_pos.T @ W_edge[:, :NUM_POS].T)
    bias1 = (W_edge[:, :NUM_POS] @ b_pos.astype(f32))[None, :]
    gam = gamma.astype(f32)[None, :]
    bet = beta.astype(f32)[None, :]

    nres = L // _RES
    wspec = lambda shape: pl.BlockSpec(shape, lambda b, i: tuple(0 for _ in shape))
    E = pl.pallas_call(
        _feat_body,
        grid=(B, nres),
        in_specs=[
            pl.BlockSpec((1, _RES, _FD), lambda b, i: (b, i, 0)),
            pl.BlockSpec((1, _RES, TOP_K, 32), lambda b, i: (b, i, 0, 0)),
            wspec((EB, _RES)),
            wspec((32, _FD)),
            wspec((1, _F)),
            wspec((128, HIDDEN)),
            wspec((_F, HIDDEN)),
            wspec((1, HIDDEN)),
            wspec((1, HIDDEN)),
            wspec((1, HIDDEN)),
        ],
        out_specs=pl.BlockSpec((1, _RES, TOP_K, HIDDEN),
                               lambda b, i: (b, i, 0, 0)),
        out_shape=jax.ShapeDtypeStruct((B, L, TOP_K, HIDDEN), f32),
    )(tabc, gathered, jnp.asarray(rmat, jnp.bfloat16), jnp.asarray(negmn, jnp.bfloat16), jnp.asarray(mu512),
      mpos, w2t, bias1, gam, bet)

    return eidx, E
